# baseline (speedup 1.0000x reference)
import jax
import jax.numpy as jnp
from jax import lax
from jax.experimental import pallas as pl
from jax.experimental.pallas import tpu as pltpu, tpu_sc as plsc

BATCH = 16384
EMBED_K = 128
CHUNK = 128
NBUF = 3
NBIG = 4

_info = plsc.get_sparse_core_info()
NC, NS = _info.num_cores, _info.num_subcores
NW = NC * NS
B_PER_W = BATCH // NW
CHUNKS_PER_W = B_PER_W // CHUNK

_mesh = plsc.VectorSubcoreMesh(core_axis_name="c", subcore_axis_name="s")


@jax.jit
def _gather2(user_idx, item_idx, user_table, item_table):
    @pl.kernel(
        mesh=_mesh,
        out_type=(
            jax.ShapeDtypeStruct((BATCH, EMBED_K), jnp.float32),
            jax.ShapeDtypeStruct((BATCH, EMBED_K), jnp.float32),
        ),
        scratch_types=[
            pltpu.VMEM((CHUNKS_PER_W, CHUNK), jnp.int32),
            pltpu.VMEM((CHUNKS_PER_W, CHUNK), jnp.int32),
            pltpu.VMEM((NBUF, 2 * CHUNK, EMBED_K), jnp.float32),
            pltpu.SemaphoreType.DMA((NBUF,)),
            pltpu.SemaphoreType.DMA((NBUF,)),
            pltpu.SemaphoreType.DMA,
        ],
    )
    def k(uidx_hbm, iidx_hbm, utab_hbm, itab_hbm, uout_hbm, iout_hbm,
          idx_u, idx_i, rows, gsem, osem, isem):
        wid = lax.axis_index("s") * NC + lax.axis_index("c")
        iu = pltpu.async_copy(
            uidx_hbm.at[pl.ds(wid * CHUNKS_PER_W, CHUNKS_PER_W)], idx_u, isem)
        ii = pltpu.async_copy(
            iidx_hbm.at[pl.ds(wid * CHUNKS_PER_W, CHUNKS_PER_W)], idx_i, isem)
        iu.wait()
        ii.wait()
        base = wid * B_PER_W

        def fire_gather(c, buf, half):
            if c < CHUNKS_PER_W:
                src = utab_hbm.at[idx_u.at[c]]
            else:
                src = itab_hbm.at[idx_i.at[c - CHUNKS_PER_W]]
            dst = rows.at[buf, pl.ds(half * CHUNK, CHUNK)]
            return pltpu.async_copy(src, dst, gsem.at[buf])

        def fire_out(b, buf):
            if b < NBIG // 2:
                dst = uout_hbm.at[pl.ds(base + b * 2 * CHUNK, 2 * CHUNK)]
            else:
                dst = iout_hbm.at[pl.ds(base + (b - NBIG // 2) * 2 * CHUNK,
                                        2 * CHUNK)]
            return pltpu.async_copy(rows.at[buf], dst, osem.at[buf])

        gathers = [[fire_gather(2 * b + h, b, h) for h in (0, 1)]
                   for b in range(NBUF)]
        outs = [None] * NBIG
        for b in range(NBIG):
            buf = b % NBUF
            gathers[buf][0].wait()
            gathers[buf][1].wait()
            outs[b] = fire_out(b, buf)
            if b + NBUF < NBIG:
                outs[b].wait()
                gathers[buf] = [fire_gather(2 * (b + NBUF) + h, buf, h)
                                for h in (0, 1)]
        for b in range(max(0, NBIG - NBUF), NBIG):
            outs[b].wait()

    return k(user_idx, item_idx, user_table, item_table)


def kernel(x, user_table, item_table):
    user_idx = x[:, 0].reshape(BATCH // CHUNK, CHUNK)
    item_idx = x[:, 1].reshape(BATCH // CHUNK, CHUNK)
    return _gather2(user_idx, item_idx, user_table, item_table)
